# baseline (speedup 1.0000x reference)
import functools

import jax
import jax.numpy as jnp
from jax import lax
from jax.experimental import pallas as pl
from jax.experimental.pallas import tpu as pltpu
from jax.experimental.pallas import tpu_sc as plsc

N = 10000
E = 320000
H = 128
NPAD = 10240

NC, NS = 2, 16
NW = NC * NS
B = 128
TB = E // B
NB_BASE = TB // NW
NB_EXTRA = TB % NW

BLK_E = 2000
GRID_E = E // BLK_E
BLK_N = 2000
GRID_N = N // BLK_N


def _p1_body(rel_ref, w1t_ref, u1c_ref, rel2_ref, rdot_ref, rmax_ref):
    rel = rel_ref[...]
    rel2_ref[...] = jnp.dot(rel, w1t_ref[...])
    rd = jnp.dot(rel, u1c_ref[...])
    rdot_ref[...] = rd
    rmax_ref[...] = jnp.max(rd).reshape(1, 1, 1)


def _p1(rel, w1t, u1c):
    return pl.pallas_call(
        _p1_body,
        grid=(GRID_E,),
        in_specs=[
            pl.BlockSpec((BLK_E, H), lambda i: (i, 0)),
            pl.BlockSpec((H, H), lambda i: (0, 0)),
            pl.BlockSpec((H, 1), lambda i: (0, 0)),
        ],
        out_specs=[
            pl.BlockSpec((BLK_E, H), lambda i: (i, 0)),
            pl.BlockSpec((BLK_E, 1), lambda i: (i, 0)),
            pl.BlockSpec((1, 1, 1), lambda i: (i, 0, 0)),
        ],
        out_shape=[
            jax.ShapeDtypeStruct((E, H), jnp.float32),
            jax.ShapeDtypeStruct((E, 1), jnp.float32),
            jax.ShapeDtypeStruct((GRID_E, 1, 1), jnp.float32),
        ],
    )(rel, w1t, u1c)


def _p1b_body(node_ref, w2t_ref, u2c_ref, nt2_ref, ndot_ref, nmax_ref):
    node = node_ref[...]
    nt2_ref[...] = jnp.dot(node, w2t_ref[...])
    nd = jnp.dot(node, u2c_ref[...])
    ndot_ref[...] = nd
    nmax_ref[...] = jnp.max(nd).reshape(1, 1, 1)


def _p1b(node, w2t, u2c):
    return pl.pallas_call(
        _p1b_body,
        grid=(GRID_N,),
        in_specs=[
            pl.BlockSpec((BLK_N, H), lambda i: (i, 0)),
            pl.BlockSpec((H, H), lambda i: (0, 0)),
            pl.BlockSpec((H, 1), lambda i: (0, 0)),
        ],
        out_specs=[
            pl.BlockSpec((BLK_N, H), lambda i: (i, 0)),
            pl.BlockSpec((BLK_N, 1), lambda i: (i, 0)),
            pl.BlockSpec((1, 1, 1), lambda i: (i, 0, 0)),
        ],
        out_shape=[
            jax.ShapeDtypeStruct((N, H), jnp.float32),
            jax.ShapeDtypeStruct((N, 1), jnp.float32),
            jax.ShapeDtypeStruct((GRID_N, 1, 1), jnp.float32),
        ],
    )(node, w2t, u2c)


def _worker_batches(c, s):
    w = s * NC + c
    nb = NB_BASE + jnp.where(w < NB_EXTRA, 1, 0)
    base = (NB_BASE * w + jnp.minimum(w, NB_EXTRA)) * B
    return w, nb, base


def _p2_body(tails_hbm, heads_hbm, rdot_hbm, ndot_hbm, cvt_hbm, shift_hbm,
             expl_hbm, segp_hbm,
             ndot_v, cvt_v, tb, hb, rb, eb, shv, seg_sh, zb):
    c = lax.axis_index("c")
    s = lax.axis_index("s")
    _, nb, base = _worker_batches(c, s)

    zeros16 = jnp.zeros((16,), jnp.float32)
    for i in range(40):
        zb[pl.ds(i * 16, 16)] = zeros16
    pltpu.sync_copy(zb, seg_sh.at[pl.ds(s * 640, 640)])

    pltpu.sync_copy(ndot_hbm, ndot_v)
    pltpu.sync_copy(cvt_hbm, cvt_v)
    pltpu.sync_copy(shift_hbm, shv)
    shift = shv[pl.ds(0, 16)][0]
    plsc.subcore_barrier()

    def batch(j, _):
        off = base + j * B
        pltpu.sync_copy(tails_hbm.at[pl.ds(off, B)], tb)
        pltpu.sync_copy(heads_hbm.at[pl.ds(off, B)], hb)
        pltpu.sync_copy(rdot_hbm.at[pl.ds(off, B)], rb)
        for g in range(B // 16):
            h16 = hb[pl.ds(g * 16, 16)]
            t16 = tb[pl.ds(g * 16, 16)]
            r16 = rb[pl.ds(g * 16, 16)]
            nd = plsc.load_gather(ndot_v, [h16])
            m = plsc.load_gather(cvt_v, [t16])
            eb[pl.ds(g * 16, 16)] = jnp.exp(r16 + nd - shift) * m
        pltpu.sync_copy(eb, expl_hbm.at[pl.ds(off, B)])
        pltpu.sync_copy(eb, seg_sh.at[tb], add=True)
        return _

    lax.fori_loop(0, nb, batch, None)
    plsc.subcore_barrier()

    @pl.when(s == 0)
    def _():
        pltpu.sync_copy(seg_sh, segp_hbm.at[c])


def _p2(tails, heads, rdot, ndot, cvtf, shift8):
    mesh = plsc.VectorSubcoreMesh(core_axis_name="c", subcore_axis_name="s",
                                  num_cores=NC, num_subcores=NS)
    f = pl.kernel(
        _p2_body,
        out_type=[
            jax.ShapeDtypeStruct((E,), jnp.float32),
            jax.ShapeDtypeStruct((NC, NPAD), jnp.float32),
        ],
        mesh=mesh,
        compiler_params=pltpu.CompilerParams(needs_layout_passes=False),
        scratch_types=[
            pltpu.VMEM((NPAD,), jnp.float32),
            pltpu.VMEM((NPAD,), jnp.float32),
            pltpu.VMEM((B,), jnp.int32),
            pltpu.VMEM((B,), jnp.int32),
            pltpu.VMEM((B,), jnp.float32),
            pltpu.VMEM((B,), jnp.float32),
            pltpu.VMEM((128,), jnp.float32),
            pltpu.VMEM_SHARED((NPAD,), jnp.float32),
            pltpu.VMEM((640,), jnp.float32),
        ],
    )
    return f(tails, heads, rdot, ndot, cvtf, shift8)


B3 = 128
TB3 = E // B3
NB3_BASE = TB3 // NW
NB3_EXTRA = TB3 % NW


def _p3_body(tails_hbm, heads_hbm, expl_hbm, rel2_hbm, nt2_hbm,
             sp_hbm,
             tb, hb, ev, rr, nr, s_sh):
    c = lax.axis_index("c")
    s = lax.axis_index("s")
    w = s * NC + c
    nbr = NB3_BASE + jnp.where(w < NB3_EXTRA, 1, 0)
    base = (NB3_BASE * w + jnp.minimum(w, NB3_EXTRA)) * B3

    zeros16 = jnp.zeros((16,), jnp.float32)

    def zrow_init(r, carry):
        for k in range(H // 16):
            rr[r, pl.ds(k * 16, 16)] = zeros16
        return carry

    lax.fori_loop(0, B3, zrow_init, None)
    done = 0
    while done < 640:
        step = min(B3, 640 - done)
        pltpu.sync_copy(rr.at[pl.ds(0, step)],
                        s_sh.at[pl.ds(s * 640 + done, step)])
        done += step
    plsc.subcore_barrier()

    def batch(j, carry):
        off = base + j * B3
        pltpu.sync_copy(tails_hbm.at[pl.ds(off, B3)], tb)
        pltpu.sync_copy(heads_hbm.at[pl.ds(off, B3)], hb)
        pltpu.sync_copy(expl_hbm.at[pl.ds(off, B3)], ev)
        pltpu.sync_copy(rel2_hbm.at[pl.ds(off, B3)], rr)
        pltpu.sync_copy(nt2_hbm.at[hb], nr)

        def scale(ri, c2):
            ev16 = ev[pl.ds(ri * 16, 16)]
            for li in range(16):
                row = ri * 16 + li
                sc = ev16[li]
                for k in range(H // 16):
                    d = pl.ds(k * 16, 16)
                    rr[row, d] = (rr[row, d] + nr[row, d]) * sc
            return c2

        lax.fori_loop(0, B3 // 16, scale, None)
        pltpu.sync_copy(rr, s_sh.at[tb], add=True)
        return carry

    lax.fori_loop(0, nbr, batch, None)

    plsc.subcore_barrier()
    pltpu.sync_copy(s_sh.at[pl.ds(s * 640, 640)],
                    sp_hbm.at[c, pl.ds(s * 640, 640)])


def _p3(tails, heads, expl, rel2, nt2):
    mesh = plsc.VectorSubcoreMesh(core_axis_name="c", subcore_axis_name="s",
                                  num_cores=NC, num_subcores=NS)
    f = pl.kernel(
        _p3_body,
        out_type=[
            jax.ShapeDtypeStruct((NC, NPAD, H), jnp.float32),
        ],
        mesh=mesh,
        compiler_params=pltpu.CompilerParams(needs_layout_passes=False),
        scratch_types=[
            pltpu.VMEM((B3,), jnp.int32),
            pltpu.VMEM((B3,), jnp.int32),
            pltpu.VMEM((B3,), jnp.float32),
            pltpu.VMEM((B3, H), jnp.float32),
            pltpu.VMEM((B3, H), jnp.float32),
            pltpu.VMEM_SHARED((NPAD, H), jnp.float32),
        ],
    )
    return f(tails, heads, expl, rel2, nt2)


def _p4_body(node_ref, sp_ref, seg_ref, cvt_ref, shc_ref, out_ref):
    seg = seg_ref[0] + seg_ref[1]
    dinv = 1.0 / jnp.where(seg > 0.0, seg, 1.0)
    agg = (sp_ref[0] + sp_ref[1]) * dinv[:, None] + shc_ref[...]
    cv = cvt_ref[0]
    out_ref[...] = jnp.where(cv[:, None] > 0.0, agg, node_ref[...])


BLK_P4 = 2048
GRID_P4 = NPAD // BLK_P4


def _p4(node, sp, segp, cvt2d, shc2d):
    return pl.pallas_call(
        _p4_body,
        grid=(GRID_P4,),
        in_specs=[
            pl.BlockSpec((BLK_P4, H), lambda i: (i, 0)),
            pl.BlockSpec((NC, BLK_P4, H), lambda i: (0, i, 0)),
            pl.BlockSpec((NC, BLK_P4), lambda i: (0, i)),
            pl.BlockSpec((1, BLK_P4), lambda i: (0, i)),
            pl.BlockSpec((1, H), lambda i: (0, 0)),
        ],
        out_specs=pl.BlockSpec((BLK_P4, H), lambda i: (i, 0)),
        out_shape=jax.ShapeDtypeStruct((N, H), jnp.float32),
    )(node, sp, segp, cvt2d, shc2d)


@jax.jit
def kernel(node_tokens, relation_tokens, edge_index, node_is_cvt, W,
           attn_vector, shared_cvt):
    heads = edge_index[0]
    tails = edge_index[1]
    cvtf = node_is_cvt.astype(jnp.float32)
    w1t = W[:, :H].T
    w2t = W[:, H:].T
    u1c = jnp.dot(w1t, attn_vector).reshape(H, 1)
    u2c = jnp.dot(w2t, attn_vector).reshape(H, 1)

    rel2, rdot2, rmax3 = _p1(relation_tokens, w1t, u1c)
    nt2, ndot2, nmax3 = _p1b(node_tokens, w2t, u2c)

    shift = jnp.max(rmax3) + jnp.max(nmax3)
    shift128 = jnp.broadcast_to(shift, (128,))
    rdot = rdot2.reshape(E)
    ndot = jnp.pad(ndot2.reshape(N), (0, NPAD - N))
    cvtp = jnp.pad(cvtf, (0, NPAD - N))

    expl, segp = _p2(tails, heads, rdot, ndot, cvtp, shift128)
    (sp,) = _p3(tails, heads, expl, rel2, nt2)

    return _p4(node_tokens, sp, segp, cvtf.reshape(1, N),
               shared_cvt.reshape(1, H))


# speedup vs baseline: 1.2952x; 1.2952x over previous
import functools

import jax
import jax.numpy as jnp
from jax import lax
from jax.experimental import pallas as pl
from jax.experimental.pallas import tpu as pltpu
from jax.experimental.pallas import tpu_sc as plsc

N = 10000
E = 320000
H = 128
NPAD = 10240

NC, NS = 2, 16
NW = NC * NS
B = 128
TB = E // B
NB_BASE = TB // NW
NB_EXTRA = TB % NW

BLK_E = 2000
GRID_E = E // BLK_E
BLK_N = 2000
GRID_N = N // BLK_N


def _p1_body(rel_ref, w1t_ref, u1c_ref, rel2_ref, rdot_ref, rmax_ref):
    rel = rel_ref[...]
    rel2_ref[...] = jnp.dot(rel, w1t_ref[...])
    rd = jnp.dot(rel, u1c_ref[...])
    rdot_ref[...] = rd
    rmax_ref[...] = jnp.max(rd).reshape(1, 1, 1)


def _p1(rel, w1t, u1c):
    return pl.pallas_call(
        _p1_body,
        grid=(GRID_E,),
        in_specs=[
            pl.BlockSpec((BLK_E, H), lambda i: (i, 0)),
            pl.BlockSpec((H, H), lambda i: (0, 0)),
            pl.BlockSpec((H, 1), lambda i: (0, 0)),
        ],
        out_specs=[
            pl.BlockSpec((BLK_E, H), lambda i: (i, 0)),
            pl.BlockSpec((BLK_E, 1), lambda i: (i, 0)),
            pl.BlockSpec((1, 1, 1), lambda i: (i, 0, 0)),
        ],
        out_shape=[
            jax.ShapeDtypeStruct((E, H), jnp.float32),
            jax.ShapeDtypeStruct((E, 1), jnp.float32),
            jax.ShapeDtypeStruct((GRID_E, 1, 1), jnp.float32),
        ],
    )(rel, w1t, u1c)


def _p1b_body(node_ref, w2t_ref, u2c_ref, nt2_ref, ndot_ref, nmax_ref):
    node = node_ref[...]
    nt2_ref[...] = jnp.dot(node, w2t_ref[...])
    nd = jnp.dot(node, u2c_ref[...])
    ndot_ref[...] = nd
    nmax_ref[...] = jnp.max(nd).reshape(1, 1, 1)


def _p1b(node, w2t, u2c):
    return pl.pallas_call(
        _p1b_body,
        grid=(GRID_N,),
        in_specs=[
            pl.BlockSpec((BLK_N, H), lambda i: (i, 0)),
            pl.BlockSpec((H, H), lambda i: (0, 0)),
            pl.BlockSpec((H, 1), lambda i: (0, 0)),
        ],
        out_specs=[
            pl.BlockSpec((BLK_N, H), lambda i: (i, 0)),
            pl.BlockSpec((BLK_N, 1), lambda i: (i, 0)),
            pl.BlockSpec((1, 1, 1), lambda i: (i, 0, 0)),
        ],
        out_shape=[
            jax.ShapeDtypeStruct((N, H), jnp.float32),
            jax.ShapeDtypeStruct((N, 1), jnp.float32),
            jax.ShapeDtypeStruct((GRID_N, 1, 1), jnp.float32),
        ],
    )(node, w2t, u2c)


def _worker_batches(c, s):
    w = s * NC + c
    nb = NB_BASE + jnp.where(w < NB_EXTRA, 1, 0)
    base = (NB_BASE * w + jnp.minimum(w, NB_EXTRA)) * B
    return w, nb, base


def _p2_body(tails_hbm, heads_hbm, rdot_hbm, ndot_hbm, cvt_hbm, shift_hbm,
             expl_hbm, segp_hbm,
             ndot_v, cvt_v, tb, hb, rb, eb, shv, seg_sh, zb):
    c = lax.axis_index("c")
    s = lax.axis_index("s")
    _, nb, base = _worker_batches(c, s)

    zeros16 = jnp.zeros((16,), jnp.float32)
    for i in range(40):
        zb[pl.ds(i * 16, 16)] = zeros16
    pltpu.sync_copy(zb, seg_sh.at[pl.ds(s * 640, 640)])

    pltpu.sync_copy(ndot_hbm, ndot_v)
    pltpu.sync_copy(cvt_hbm, cvt_v)
    pltpu.sync_copy(shift_hbm, shv)
    shift = shv[pl.ds(0, 16)][0]
    plsc.subcore_barrier()

    def batch(j, _):
        off = base + j * B
        pltpu.sync_copy(tails_hbm.at[pl.ds(off, B)], tb)
        pltpu.sync_copy(heads_hbm.at[pl.ds(off, B)], hb)
        pltpu.sync_copy(rdot_hbm.at[pl.ds(off, B)], rb)
        for g in range(B // 16):
            h16 = hb[pl.ds(g * 16, 16)]
            t16 = tb[pl.ds(g * 16, 16)]
            r16 = rb[pl.ds(g * 16, 16)]
            nd = plsc.load_gather(ndot_v, [h16])
            m = plsc.load_gather(cvt_v, [t16])
            eb[pl.ds(g * 16, 16)] = jnp.exp(r16 + nd - shift) * m
        pltpu.sync_copy(eb, expl_hbm.at[pl.ds(off, B)])
        pltpu.sync_copy(eb, seg_sh.at[tb], add=True)
        return _

    lax.fori_loop(0, nb, batch, None)
    plsc.subcore_barrier()

    @pl.when(s == 0)
    def _():
        pltpu.sync_copy(seg_sh, segp_hbm.at[c])


def _p2(tails, heads, rdot, ndot, cvtf, shift8):
    mesh = plsc.VectorSubcoreMesh(core_axis_name="c", subcore_axis_name="s",
                                  num_cores=NC, num_subcores=NS)
    f = pl.kernel(
        _p2_body,
        out_type=[
            jax.ShapeDtypeStruct((E,), jnp.float32),
            jax.ShapeDtypeStruct((NC, NPAD), jnp.float32),
        ],
        mesh=mesh,
        compiler_params=pltpu.CompilerParams(needs_layout_passes=False),
        scratch_types=[
            pltpu.VMEM((NPAD,), jnp.float32),
            pltpu.VMEM((NPAD,), jnp.float32),
            pltpu.VMEM((B,), jnp.int32),
            pltpu.VMEM((B,), jnp.int32),
            pltpu.VMEM((B,), jnp.float32),
            pltpu.VMEM((B,), jnp.float32),
            pltpu.VMEM((128,), jnp.float32),
            pltpu.VMEM_SHARED((NPAD,), jnp.float32),
            pltpu.VMEM((640,), jnp.float32),
        ],
    )
    return f(tails, heads, rdot, ndot, cvtf, shift8)


B3 = 64
TB3 = E // B3
NB3_BASE = TB3 // NW
NB3_EXTRA = TB3 % NW
NBV3 = 158


def _p3_body(tails_hbm, heads_hbm, expl_hbm, rel2_hbm, nt2_hbm,
             sp_hbm,
             tb, hb, ev, rr, nr, s_sh,
             sl0, sl1, sr0, sr1, sn0, sn1):
    c = lax.axis_index("c")
    s = lax.axis_index("s")
    w = s * NC + c
    nbr = NB3_BASE + jnp.where(w < NB3_EXTRA, 1, 0)
    base = (NB3_BASE * w + jnp.minimum(w, NB3_EXTRA)) * B3
    sl = [sl0, sl1]
    sr = [sr0, sr1]
    sn = [sn0, sn1]

    def off_of(j):
        return base + jnp.minimum(j, nbr - 1) * B3

    zeros16 = jnp.zeros((16,), jnp.float32)

    def zrow_init(r, carry):
        for k in range(H // 16):
            rr[0, r, pl.ds(k * 16, 16)] = zeros16
        return carry

    lax.fori_loop(0, B3, zrow_init, None)
    done = 0
    while done < 640:
        step = min(B3, 640 - done)
        pltpu.sync_copy(rr.at[0, pl.ds(0, step)],
                        s_sh.at[pl.ds(s * 640 + done, step)])
        done += step
    plsc.subcore_barrier()

    def issue_small(j, q):
        off = off_of(j)
        pltpu.async_copy(tails_hbm.at[pl.ds(off, B3)], tb.at[q], sl[q])
        pltpu.async_copy(heads_hbm.at[pl.ds(off, B3)], hb.at[q], sl[q])
        pltpu.async_copy(expl_hbm.at[pl.ds(off, B3)], ev.at[q], sl[q])

    def wait_small(j, q):
        off = off_of(j)
        pltpu.make_async_copy(tails_hbm.at[pl.ds(off, B3)], tb.at[q], sl[q]).wait()
        pltpu.make_async_copy(heads_hbm.at[pl.ds(off, B3)], hb.at[q], sl[q]).wait()
        pltpu.make_async_copy(expl_hbm.at[pl.ds(off, B3)], ev.at[q], sl[q]).wait()

    def issue_rr(j, q):
        pltpu.async_copy(rel2_hbm.at[pl.ds(off_of(j), B3)], rr.at[q], sr[q])

    def wait_rr(j, q):
        pltpu.make_async_copy(rel2_hbm.at[pl.ds(off_of(j), B3)], rr.at[q],
                              sr[q]).wait()

    def issue_nr(q):
        pltpu.async_copy(nt2_hbm.at[hb.at[q]], nr.at[q], sn[q])

    def wait_nr(q):
        pltpu.make_async_copy(nt2_hbm.at[hb.at[q]], nr.at[q], sn[q]).wait()

    def compute(j, q):
        @pl.when(j >= nbr)
        def _():
            for g in range(B3 // 16):
                ev[q, pl.ds(g * 16, 16)] = zeros16

        def scale(ri, c2):
            ev16 = ev[q, pl.ds(ri * 16, 16)]
            for li in range(16):
                row = ri * 16 + li
                sc = ev16[li]
                for k in range(H // 16):
                    d = pl.ds(k * 16, 16)
                    rr[q, row, d] = (rr[q, row, d] + nr[q, row, d]) * sc
            return c2

        lax.fori_loop(0, B3 // 16, scale, None)

    issue_small(0, 0)
    issue_rr(0, 0)
    issue_small(1, 1)
    issue_rr(1, 1)
    wait_small(0, 0)
    issue_nr(0)

    def pair(i, carry):
        for t in range(2):
            j = i * 2 + t
            q, q1 = t, 1 - t
            wait_small(j + 1, q1)
            issue_nr(q1)
            wait_rr(j, q)
            wait_nr(q)
            compute(j, q)
            pltpu.sync_copy(rr.at[q], s_sh.at[tb.at[q]], add=True)
            issue_small(j + 2, q)
            issue_rr(j + 2, q)
        return carry

    lax.fori_loop(0, NBV3 // 2, pair, None)

    wait_small(NBV3 + 1, 1)
    wait_rr(NBV3, 0)
    wait_rr(NBV3 + 1, 1)
    wait_nr(0)

    plsc.subcore_barrier()
    pltpu.sync_copy(s_sh.at[pl.ds(s * 640, 640)],
                    sp_hbm.at[c, pl.ds(s * 640, 640)])


def _p3(tails, heads, expl, rel2, nt2):
    mesh = plsc.VectorSubcoreMesh(core_axis_name="c", subcore_axis_name="s",
                                  num_cores=NC, num_subcores=NS)
    f = pl.kernel(
        _p3_body,
        out_type=[
            jax.ShapeDtypeStruct((NC, NPAD, H), jnp.float32),
        ],
        mesh=mesh,
        compiler_params=pltpu.CompilerParams(needs_layout_passes=False),
        scratch_types=[
            pltpu.VMEM((2, B3), jnp.int32),
            pltpu.VMEM((2, B3), jnp.int32),
            pltpu.VMEM((2, B3), jnp.float32),
            pltpu.VMEM((2, B3, H), jnp.float32),
            pltpu.VMEM((2, B3, H), jnp.float32),
            pltpu.VMEM_SHARED((NPAD, H), jnp.float32),
        ] + [pltpu.SemaphoreType.DMA] * 6,
    )
    return f(tails, heads, expl, rel2, nt2)


def _p4_body(node_ref, sp_ref, seg_ref, cvt_ref, shc_ref, out_ref):
    seg = seg_ref[0] + seg_ref[1]
    dinv = 1.0 / jnp.where(seg > 0.0, seg, 1.0)
    agg = (sp_ref[0] + sp_ref[1]) * dinv[:, None] + shc_ref[...]
    cv = cvt_ref[0]
    out_ref[...] = jnp.where(cv[:, None] > 0.0, agg, node_ref[...])


BLK_P4 = 2048
GRID_P4 = NPAD // BLK_P4


def _p4(node, sp, segp, cvt2d, shc2d):
    return pl.pallas_call(
        _p4_body,
        grid=(GRID_P4,),
        in_specs=[
            pl.BlockSpec((BLK_P4, H), lambda i: (i, 0)),
            pl.BlockSpec((NC, BLK_P4, H), lambda i: (0, i, 0)),
            pl.BlockSpec((NC, BLK_P4), lambda i: (0, i)),
            pl.BlockSpec((1, BLK_P4), lambda i: (0, i)),
            pl.BlockSpec((1, H), lambda i: (0, 0)),
        ],
        out_specs=pl.BlockSpec((BLK_P4, H), lambda i: (i, 0)),
        out_shape=jax.ShapeDtypeStruct((N, H), jnp.float32),
    )(node, sp, segp, cvt2d, shc2d)


@jax.jit
def kernel(node_tokens, relation_tokens, edge_index, node_is_cvt, W,
           attn_vector, shared_cvt):
    heads = edge_index[0]
    tails = edge_index[1]
    cvtf = node_is_cvt.astype(jnp.float32)
    w1t = W[:, :H].T
    w2t = W[:, H:].T
    u1c = jnp.dot(w1t, attn_vector).reshape(H, 1)
    u2c = jnp.dot(w2t, attn_vector).reshape(H, 1)

    rel2, rdot2, rmax3 = _p1(relation_tokens, w1t, u1c)
    nt2, ndot2, nmax3 = _p1b(node_tokens, w2t, u2c)

    shift = jnp.max(rmax3) + jnp.max(nmax3)
    shift128 = jnp.broadcast_to(shift, (128,))
    rdot = rdot2.reshape(E)
    ndot = jnp.pad(ndot2.reshape(N), (0, NPAD - N))
    cvtp = jnp.pad(cvtf, (0, NPAD - N))

    expl, segp = _p2(tails, heads, rdot, ndot, cvtp, shift128)
    (sp,) = _p3(tails, heads, expl, rel2, nt2)

    return _p4(node_tokens, sp, segp, cvtf.reshape(1, N),
               shared_cvt.reshape(1, H))


# speedup vs baseline: 1.4916x; 1.1516x over previous
import functools

import jax
import jax.numpy as jnp
from jax import lax
from jax.experimental import pallas as pl
from jax.experimental.pallas import tpu as pltpu
from jax.experimental.pallas import tpu_sc as plsc

N = 10000
E = 320000
H = 128
NPAD = 10240

NC, NS = 2, 16
NW = NC * NS
B = 128
TB = E // B
NB_BASE = TB // NW
NB_EXTRA = TB % NW

BLK_E = 2000
GRID_E = E // BLK_E
BLK_N = 2000
GRID_N = N // BLK_N


def _p1_body(rel_ref, w1t_ref, u1c_ref, rel2_ref, rdot_ref, rmax_ref):
    rel = rel_ref[...]
    rel2_ref[...] = jnp.dot(rel, w1t_ref[...])
    rd = jnp.dot(rel, u1c_ref[...])
    rdot_ref[...] = rd
    rmax_ref[...] = jnp.max(rd).reshape(1, 1, 1)


def _p1(rel, w1t, u1c):
    return pl.pallas_call(
        _p1_body,
        grid=(GRID_E,),
        in_specs=[
            pl.BlockSpec((BLK_E, H), lambda i: (i, 0)),
            pl.BlockSpec((H, H), lambda i: (0, 0)),
            pl.BlockSpec((H, 1), lambda i: (0, 0)),
        ],
        out_specs=[
            pl.BlockSpec((BLK_E, H), lambda i: (i, 0)),
            pl.BlockSpec((BLK_E, 1), lambda i: (i, 0)),
            pl.BlockSpec((1, 1, 1), lambda i: (i, 0, 0)),
        ],
        out_shape=[
            jax.ShapeDtypeStruct((E, H), jnp.float32),
            jax.ShapeDtypeStruct((E, 1), jnp.float32),
            jax.ShapeDtypeStruct((GRID_E, 1, 1), jnp.float32),
        ],
    )(rel, w1t, u1c)


def _p1b_body(node_ref, w2t_ref, u2c_ref, nt2_ref, ndot_ref, nmax_ref):
    node = node_ref[...]
    nt2_ref[...] = jnp.dot(node, w2t_ref[...])
    nd = jnp.dot(node, u2c_ref[...])
    ndot_ref[...] = nd
    nmax_ref[...] = jnp.max(nd).reshape(1, 1, 1)


def _p1b(node, w2t, u2c):
    return pl.pallas_call(
        _p1b_body,
        grid=(GRID_N,),
        in_specs=[
            pl.BlockSpec((BLK_N, H), lambda i: (i, 0)),
            pl.BlockSpec((H, H), lambda i: (0, 0)),
            pl.BlockSpec((H, 1), lambda i: (0, 0)),
        ],
        out_specs=[
            pl.BlockSpec((BLK_N, H), lambda i: (i, 0)),
            pl.BlockSpec((BLK_N, 1), lambda i: (i, 0)),
            pl.BlockSpec((1, 1, 1), lambda i: (i, 0, 0)),
        ],
        out_shape=[
            jax.ShapeDtypeStruct((N, H), jnp.float32),
            jax.ShapeDtypeStruct((N, 1), jnp.float32),
            jax.ShapeDtypeStruct((GRID_N, 1, 1), jnp.float32),
        ],
    )(node, w2t, u2c)


B3 = 64
TB3 = E // B3
NB3_BASE = TB3 // NW
NB3_EXTRA = TB3 % NW
NBV3 = 158


def _p23_body(tails_hbm, heads_hbm, rdot_hbm, ndot_hbm, cvt_hbm, shift_hbm,
              rel2_hbm, nt2_hbm,
              sp_hbm, segp_hbm,
              tb, hb, rd, nd, cv, ev, rr, nr, shv, s_sh, g_sh,
              sl0, sl1, sr0, sr1, sn0, sn1, sg0, sg1):
    c = lax.axis_index("c")
    s = lax.axis_index("s")
    w = s * NC + c
    nbr = NB3_BASE + jnp.where(w < NB3_EXTRA, 1, 0)
    base = (NB3_BASE * w + jnp.minimum(w, NB3_EXTRA)) * B3
    sl = [sl0, sl1]
    sr = [sr0, sr1]
    sn = [sn0, sn1]
    sg = [sg0, sg1]

    def off_of(j):
        return base + jnp.minimum(j, nbr - 1) * B3

    zeros16 = jnp.zeros((16,), jnp.float32)

    def zrow_init(r, carry):
        for k in range(H // 16):
            rr[0, r, pl.ds(k * 16, 16)] = zeros16
        return carry

    lax.fori_loop(0, B3, zrow_init, None)
    for g in range(B3 // 16):
        ev[0, pl.ds(g * 16, 16)] = zeros16
    for i in range(640 // B3):
        pltpu.sync_copy(rr.at[0, pl.ds(0, B3)],
                        s_sh.at[pl.ds(s * 640 + i * B3, B3)])
        pltpu.sync_copy(ev.at[0],
                        g_sh.at[pl.ds(s * 640 + i * B3, B3)])
    pltpu.sync_copy(shift_hbm, shv)
    shift = shv[pl.ds(0, 16)][0]
    plsc.subcore_barrier()

    def issue_small(j, q):
        off = off_of(j)
        pltpu.async_copy(tails_hbm.at[pl.ds(off, B3)], tb.at[q], sl[q])
        pltpu.async_copy(heads_hbm.at[pl.ds(off, B3)], hb.at[q], sl[q])
        pltpu.async_copy(rdot_hbm.at[pl.ds(off, B3)], rd.at[q], sl[q])

    def wait_small(j, q):
        off = off_of(j)
        pltpu.make_async_copy(tails_hbm.at[pl.ds(off, B3)], tb.at[q], sl[q]).wait()
        pltpu.make_async_copy(heads_hbm.at[pl.ds(off, B3)], hb.at[q], sl[q]).wait()
        pltpu.make_async_copy(rdot_hbm.at[pl.ds(off, B3)], rd.at[q], sl[q]).wait()

    def issue_gath(q):
        pltpu.async_copy(ndot_hbm.at[hb.at[q]], nd.at[q], sg[q])
        pltpu.async_copy(cvt_hbm.at[tb.at[q]], cv.at[q], sg[q])

    def wait_gath(q):
        pltpu.make_async_copy(ndot_hbm.at[hb.at[q]], nd.at[q], sg[q]).wait()
        pltpu.make_async_copy(cvt_hbm.at[tb.at[q]], cv.at[q], sg[q]).wait()

    def issue_rr(j, q):
        pltpu.async_copy(rel2_hbm.at[pl.ds(off_of(j), B3)], rr.at[q], sr[q])

    def wait_rr(j, q):
        pltpu.make_async_copy(rel2_hbm.at[pl.ds(off_of(j), B3)], rr.at[q],
                              sr[q]).wait()

    def issue_nr(q):
        pltpu.async_copy(nt2_hbm.at[hb.at[q]], nr.at[q], sn[q])

    def wait_nr(q):
        pltpu.make_async_copy(nt2_hbm.at[hb.at[q]], nr.at[q], sn[q]).wait()

    def compute(j, q):
        for g in range(B3 // 16):
            d = pl.ds(g * 16, 16)
            ev[q, d] = jnp.exp(rd[q, d] + nd[q, d] - shift) * cv[q, d]

        @pl.when(j >= nbr)
        def _():
            for g in range(B3 // 16):
                ev[q, pl.ds(g * 16, 16)] = zeros16

        def scale(ri, c2):
            ev16 = ev[q, pl.ds(ri * 16, 16)]
            for li in range(16):
                row = ri * 16 + li
                sc = ev16[li]
                for k in range(H // 16):
                    d = pl.ds(k * 16, 16)
                    rr[q, row, d] = (rr[q, row, d] + nr[q, row, d]) * sc
            return c2

        lax.fori_loop(0, B3 // 16, scale, None)

    issue_small(0, 0)
    issue_rr(0, 0)
    issue_small(1, 1)
    issue_rr(1, 1)
    wait_small(0, 0)
    issue_gath(0)
    issue_nr(0)

    def pair(i, carry):
        for t in range(2):
            j = i * 2 + t
            q, q1 = t, 1 - t
            wait_small(j + 1, q1)
            issue_gath(q1)
            issue_nr(q1)
            wait_rr(j, q)
            wait_nr(q)
            wait_gath(q)
            compute(j, q)
            pltpu.sync_copy(ev.at[q], g_sh.at[tb.at[q]], add=True)
            pltpu.sync_copy(rr.at[q], s_sh.at[tb.at[q]], add=True)
            issue_small(j + 2, q)
            issue_rr(j + 2, q)
        return carry

    lax.fori_loop(0, NBV3 // 2, pair, None)

    wait_small(NBV3 + 1, 1)
    wait_rr(NBV3, 0)
    wait_rr(NBV3 + 1, 1)
    wait_nr(0)
    wait_gath(0)

    plsc.subcore_barrier()
    pltpu.sync_copy(s_sh.at[pl.ds(s * 640, 640)],
                    sp_hbm.at[c, pl.ds(s * 640, 640)])
    pltpu.sync_copy(g_sh.at[pl.ds(s * 640, 640)],
                    segp_hbm.at[c, pl.ds(s * 640, 640)])


def _p23(tails, heads, rdot, ndot, cvtf, shift128, rel2, nt2):
    mesh = plsc.VectorSubcoreMesh(core_axis_name="c", subcore_axis_name="s",
                                  num_cores=NC, num_subcores=NS)
    f = pl.kernel(
        _p23_body,
        out_type=[
            jax.ShapeDtypeStruct((NC, NPAD, H), jnp.float32),
            jax.ShapeDtypeStruct((NC, NPAD), jnp.float32),
        ],
        mesh=mesh,
        compiler_params=pltpu.CompilerParams(needs_layout_passes=False),
        scratch_types=[
            pltpu.VMEM((2, B3), jnp.int32),
            pltpu.VMEM((2, B3), jnp.int32),
            pltpu.VMEM((2, B3), jnp.float32),
            pltpu.VMEM((2, B3), jnp.float32),
            pltpu.VMEM((2, B3), jnp.float32),
            pltpu.VMEM((2, B3), jnp.float32),
            pltpu.VMEM((2, B3, H), jnp.float32),
            pltpu.VMEM((2, B3, H), jnp.float32),
            pltpu.VMEM((128,), jnp.float32),
            pltpu.VMEM_SHARED((NPAD, H), jnp.float32),
            pltpu.VMEM_SHARED((NPAD,), jnp.float32),
        ] + [pltpu.SemaphoreType.DMA] * 8,
    )
    return f(tails, heads, rdot, ndot, cvtf, shift128, rel2, nt2)


def _p4_body(node_ref, sp_ref, seg_ref, cvt_ref, shc_ref, out_ref):
    seg = seg_ref[0] + seg_ref[1]
    dinv = 1.0 / jnp.where(seg > 0.0, seg, 1.0)
    agg = (sp_ref[0] + sp_ref[1]) * dinv[:, None] + shc_ref[...]
    cv = cvt_ref[0]
    out_ref[...] = jnp.where(cv[:, None] > 0.0, agg, node_ref[...])


BLK_P4 = 2048
GRID_P4 = NPAD // BLK_P4


def _p4(node, sp, segp, cvt2d, shc2d):
    return pl.pallas_call(
        _p4_body,
        grid=(GRID_P4,),
        in_specs=[
            pl.BlockSpec((BLK_P4, H), lambda i: (i, 0)),
            pl.BlockSpec((NC, BLK_P4, H), lambda i: (0, i, 0)),
            pl.BlockSpec((NC, BLK_P4), lambda i: (0, i)),
            pl.BlockSpec((1, BLK_P4), lambda i: (0, i)),
            pl.BlockSpec((1, H), lambda i: (0, 0)),
        ],
        out_specs=pl.BlockSpec((BLK_P4, H), lambda i: (i, 0)),
        out_shape=jax.ShapeDtypeStruct((N, H), jnp.float32),
    )(node, sp, segp, cvt2d, shc2d)


@jax.jit
def kernel(node_tokens, relation_tokens, edge_index, node_is_cvt, W,
           attn_vector, shared_cvt):
    heads = edge_index[0]
    tails = edge_index[1]
    cvtf = node_is_cvt.astype(jnp.float32)
    w1t = W[:, :H].T
    w2t = W[:, H:].T
    u1c = jnp.dot(w1t, attn_vector).reshape(H, 1)
    u2c = jnp.dot(w2t, attn_vector).reshape(H, 1)

    rel2, rdot2, rmax3 = _p1(relation_tokens, w1t, u1c)
    nt2, ndot2, nmax3 = _p1b(node_tokens, w2t, u2c)

    shift = jnp.max(rmax3) + jnp.max(nmax3)
    shift128 = jnp.broadcast_to(shift, (128,))
    rdot = rdot2.reshape(E)
    ndot = ndot2.reshape(N)

    sp, segp = _p23(tails, heads, rdot, ndot, cvtf, shift128, rel2, nt2)

    return _p4(node_tokens, sp, segp, cvtf.reshape(1, N),
               shared_cvt.reshape(1, H))


# speedup vs baseline: 1.6106x; 1.0798x over previous
import functools

import jax
import jax.numpy as jnp
from jax import lax
from jax.experimental import pallas as pl
from jax.experimental.pallas import tpu as pltpu
from jax.experimental.pallas import tpu_sc as plsc

N = 10000
E = 320000
H = 128
NPAD = 10240

NC, NS = 2, 16
NW = NC * NS
B = 128
TB = E // B
NB_BASE = TB // NW
NB_EXTRA = TB % NW

BLK_E = 2000
GRID_E = E // BLK_E
BLK_N = 2000
GRID_N = N // BLK_N


def _p1_body(rel_ref, w1t_ref, u1c_ref, rel2_ref, rdot_ref, rmax_ref):
    rel = rel_ref[...]
    rel2_ref[...] = jnp.dot(rel, w1t_ref[...])
    rd = jnp.dot(rel, u1c_ref[...])
    rdot_ref[...] = rd
    rmax_ref[...] = jnp.max(rd).reshape(1, 1, 1)


def _p1(rel, w1t, u1c):
    return pl.pallas_call(
        _p1_body,
        grid=(GRID_E,),
        in_specs=[
            pl.BlockSpec((BLK_E, H), lambda i: (i, 0)),
            pl.BlockSpec((H, H), lambda i: (0, 0)),
            pl.BlockSpec((H, 1), lambda i: (0, 0)),
        ],
        out_specs=[
            pl.BlockSpec((BLK_E, H), lambda i: (i, 0)),
            pl.BlockSpec((BLK_E, 1), lambda i: (i, 0)),
            pl.BlockSpec((1, 1, 1), lambda i: (i, 0, 0)),
        ],
        out_shape=[
            jax.ShapeDtypeStruct((E, H), jnp.float32),
            jax.ShapeDtypeStruct((E, 1), jnp.float32),
            jax.ShapeDtypeStruct((GRID_E, 1, 1), jnp.float32),
        ],
    )(rel, w1t, u1c)


def _p1b_body(node_ref, w2t_ref, u2c_ref, nt2_ref, ndot_ref, nmax_ref):
    node = node_ref[...]
    nt2_ref[...] = jnp.dot(node, w2t_ref[...])
    nd = jnp.dot(node, u2c_ref[...])
    ndot_ref[...] = nd
    nmax_ref[...] = jnp.max(nd).reshape(1, 1, 1)


def _p1b(node, w2t, u2c):
    return pl.pallas_call(
        _p1b_body,
        grid=(GRID_N,),
        in_specs=[
            pl.BlockSpec((BLK_N, H), lambda i: (i, 0)),
            pl.BlockSpec((H, H), lambda i: (0, 0)),
            pl.BlockSpec((H, 1), lambda i: (0, 0)),
        ],
        out_specs=[
            pl.BlockSpec((BLK_N, H), lambda i: (i, 0)),
            pl.BlockSpec((BLK_N, 1), lambda i: (i, 0)),
            pl.BlockSpec((1, 1, 1), lambda i: (i, 0, 0)),
        ],
        out_shape=[
            jax.ShapeDtypeStruct((N, H), jnp.float32),
            jax.ShapeDtypeStruct((N, 1), jnp.float32),
            jax.ShapeDtypeStruct((GRID_N, 1, 1), jnp.float32),
        ],
    )(node, w2t, u2c)


B3 = 64
TB3 = E // B3
NB3_BASE = TB3 // NW
NB3_EXTRA = TB3 % NW
NBV3 = 158


def _p23_body(tails_hbm, heads_hbm, rdot_hbm, ndot_hbm, cvt_hbm, shift_hbm,
              rel2_hbm, nt2_hbm,
              sp_hbm, segp_hbm,
              tb, hb, rd, nd, cv, ev, rr, nr, shv, s_sh, g_sh,
              sl0, sl1, sr0, sr1, sn0, sn1, sg0, sg1):
    c = lax.axis_index("c")
    s = lax.axis_index("s")
    w = s * NC + c
    nbr = NB3_BASE + jnp.where(w < NB3_EXTRA, 1, 0)
    base = (NB3_BASE * w + jnp.minimum(w, NB3_EXTRA)) * B3
    sl = [sl0, sl1]
    sr = [sr0, sr1]
    sn = [sn0, sn1]
    sg = [sg0, sg1]

    def off_of(j):
        return base + jnp.minimum(j, nbr - 1) * B3

    zeros16 = jnp.zeros((16,), jnp.float32)

    def zrow_init(r, carry):
        for k in range(H // 16):
            rr[0, r, pl.ds(k * 16, 16)] = zeros16
        return carry

    lax.fori_loop(0, B3, zrow_init, None)
    for g in range(B3 // 16):
        ev[0, pl.ds(g * 16, 16)] = zeros16
    for i in range(640 // B3):
        pltpu.sync_copy(rr.at[0, pl.ds(0, B3)],
                        s_sh.at[pl.ds(s * 640 + i * B3, B3)])
        pltpu.sync_copy(ev.at[0],
                        g_sh.at[pl.ds(s * 640 + i * B3, B3)])
    pltpu.sync_copy(shift_hbm, shv)
    shift = shv[pl.ds(0, 16)][0]
    plsc.subcore_barrier()

    def issue_small(j, q):
        off = off_of(j)
        pltpu.async_copy(tails_hbm.at[pl.ds(off, B3)], tb.at[q], sl[q])
        pltpu.async_copy(heads_hbm.at[pl.ds(off, B3)], hb.at[q], sl[q])
        pltpu.async_copy(rdot_hbm.at[pl.ds(off, B3)], rd.at[q], sl[q])

    def wait_small(j, q):
        off = off_of(j)
        pltpu.make_async_copy(tails_hbm.at[pl.ds(off, B3)], tb.at[q], sl[q]).wait()
        pltpu.make_async_copy(heads_hbm.at[pl.ds(off, B3)], hb.at[q], sl[q]).wait()
        pltpu.make_async_copy(rdot_hbm.at[pl.ds(off, B3)], rd.at[q], sl[q]).wait()

    def issue_gath(q):
        pltpu.async_copy(ndot_hbm.at[hb.at[q]], nd.at[q], sg[q])
        pltpu.async_copy(cvt_hbm.at[tb.at[q]], cv.at[q], sg[q])

    def wait_gath(q):
        pltpu.make_async_copy(ndot_hbm.at[hb.at[q]], nd.at[q], sg[q]).wait()
        pltpu.make_async_copy(cvt_hbm.at[tb.at[q]], cv.at[q], sg[q]).wait()

    def issue_rr(j, q):
        pltpu.async_copy(rel2_hbm.at[pl.ds(off_of(j), B3)], rr.at[q], sr[q])

    def wait_rr(j, q):
        pltpu.make_async_copy(rel2_hbm.at[pl.ds(off_of(j), B3)], rr.at[q],
                              sr[q]).wait()

    def issue_nr(q):
        pltpu.async_copy(nt2_hbm.at[hb.at[q]], nr.at[q], sn[q])

    def wait_nr(q):
        pltpu.make_async_copy(nt2_hbm.at[hb.at[q]], nr.at[q], sn[q]).wait()

    def compute(j, q):
        for g in range(B3 // 16):
            d = pl.ds(g * 16, 16)
            ev[q, d] = jnp.exp(rd[q, d] + nd[q, d] - shift) * cv[q, d]

        @pl.when(j >= nbr)
        def _():
            for g in range(B3 // 16):
                ev[q, pl.ds(g * 16, 16)] = zeros16

        def scale(ri, c2):
            d16 = pl.ds(ri * 16, 16)
            ev16 = ev[q, d16]
            tb[q, d16] = jnp.where(ev16 > 0.0, tb[q, d16], N)
            for li in range(16):
                row = ri * 16 + li
                sc = ev16[li]

                @pl.when(sc > 0.0)
                def _():
                    for k in range(H // 16):
                        d = pl.ds(k * 16, 16)
                        rr[q, row, d] = (rr[q, row, d] + nr[q, row, d]) * sc
            return c2

        lax.fori_loop(0, B3 // 16, scale, None)

    issue_small(0, 0)
    issue_rr(0, 0)
    issue_small(1, 1)
    issue_rr(1, 1)
    wait_small(0, 0)
    issue_gath(0)
    issue_nr(0)

    def pair(i, carry):
        for t in range(2):
            j = i * 2 + t
            q, q1 = t, 1 - t
            wait_small(j + 1, q1)
            issue_gath(q1)
            issue_nr(q1)
            wait_rr(j, q)
            wait_nr(q)
            wait_gath(q)
            compute(j, q)
            pltpu.sync_copy(ev.at[q], g_sh.at[tb.at[q]], add=True)
            pltpu.sync_copy(rr.at[q], s_sh.at[tb.at[q]], add=True)
            issue_small(j + 2, q)
            issue_rr(j + 2, q)
        return carry

    lax.fori_loop(0, NBV3 // 2, pair, None)

    wait_small(NBV3 + 1, 1)
    wait_rr(NBV3, 0)
    wait_rr(NBV3 + 1, 1)
    wait_nr(0)
    wait_gath(0)

    plsc.subcore_barrier()
    pltpu.sync_copy(s_sh.at[pl.ds(s * 640, 640)],
                    sp_hbm.at[c, pl.ds(s * 640, 640)])
    pltpu.sync_copy(g_sh.at[pl.ds(s * 640, 640)],
                    segp_hbm.at[c, pl.ds(s * 640, 640)])


def _p23(tails, heads, rdot, ndot, cvtf, shift128, rel2, nt2):
    mesh = plsc.VectorSubcoreMesh(core_axis_name="c", subcore_axis_name="s",
                                  num_cores=NC, num_subcores=NS)
    f = pl.kernel(
        _p23_body,
        out_type=[
            jax.ShapeDtypeStruct((NC, NPAD, H), jnp.float32),
            jax.ShapeDtypeStruct((NC, NPAD), jnp.float32),
        ],
        mesh=mesh,
        compiler_params=pltpu.CompilerParams(needs_layout_passes=False),
        scratch_types=[
            pltpu.VMEM((2, B3), jnp.int32),
            pltpu.VMEM((2, B3), jnp.int32),
            pltpu.VMEM((2, B3), jnp.float32),
            pltpu.VMEM((2, B3), jnp.float32),
            pltpu.VMEM((2, B3), jnp.float32),
            pltpu.VMEM((2, B3), jnp.float32),
            pltpu.VMEM((2, B3, H), jnp.float32),
            pltpu.VMEM((2, B3, H), jnp.float32),
            pltpu.VMEM((128,), jnp.float32),
            pltpu.VMEM_SHARED((NPAD, H), jnp.float32),
            pltpu.VMEM_SHARED((NPAD,), jnp.float32),
        ] + [pltpu.SemaphoreType.DMA] * 8,
    )
    return f(tails, heads, rdot, ndot, cvtf, shift128, rel2, nt2)


def _p4_body(node_ref, sp_ref, seg_ref, cvt_ref, shc_ref, out_ref):
    seg = seg_ref[0] + seg_ref[1]
    dinv = 1.0 / jnp.where(seg > 0.0, seg, 1.0)
    agg = (sp_ref[0] + sp_ref[1]) * dinv[:, None] + shc_ref[...]
    cv = cvt_ref[0]
    out_ref[...] = jnp.where(cv[:, None] > 0.0, agg, node_ref[...])


BLK_P4 = 2048
GRID_P4 = NPAD // BLK_P4


def _p4(node, sp, segp, cvt2d, shc2d):
    return pl.pallas_call(
        _p4_body,
        grid=(GRID_P4,),
        in_specs=[
            pl.BlockSpec((BLK_P4, H), lambda i: (i, 0)),
            pl.BlockSpec((NC, BLK_P4, H), lambda i: (0, i, 0)),
            pl.BlockSpec((NC, BLK_P4), lambda i: (0, i)),
            pl.BlockSpec((1, BLK_P4), lambda i: (0, i)),
            pl.BlockSpec((1, H), lambda i: (0, 0)),
        ],
        out_specs=pl.BlockSpec((BLK_P4, H), lambda i: (i, 0)),
        out_shape=jax.ShapeDtypeStruct((N, H), jnp.float32),
    )(node, sp, segp, cvt2d, shc2d)


@jax.jit
def kernel(node_tokens, relation_tokens, edge_index, node_is_cvt, W,
           attn_vector, shared_cvt):
    heads = edge_index[0]
    tails = edge_index[1]
    cvtf = node_is_cvt.astype(jnp.float32)
    w1t = W[:, :H].T
    w2t = W[:, H:].T
    u1c = jnp.dot(w1t, attn_vector).reshape(H, 1)
    u2c = jnp.dot(w2t, attn_vector).reshape(H, 1)

    rel2, rdot2, rmax3 = _p1(relation_tokens, w1t, u1c)
    nt2, ndot2, nmax3 = _p1b(node_tokens, w2t, u2c)

    shift = jnp.max(rmax3) + jnp.max(nmax3)
    shift128 = jnp.broadcast_to(shift, (128,))
    rdot = rdot2.reshape(E)
    ndot = ndot2.reshape(N)

    sp, segp = _p23(tails, heads, rdot, ndot, cvtf, shift128, rel2, nt2)

    return _p4(node_tokens, sp, segp, cvtf.reshape(1, N),
               shared_cvt.reshape(1, H))


# speedup vs baseline: 1.7745x; 1.1017x over previous
import functools

import jax
import jax.numpy as jnp
from jax import lax
from jax.experimental import pallas as pl
from jax.experimental.pallas import tpu as pltpu
from jax.experimental.pallas import tpu_sc as plsc

N = 10000
E = 320000
H = 128
NPAD = 10240

NC, NS = 2, 16
NW = NC * NS
B = 128
TB = E // B
NB_BASE = TB // NW
NB_EXTRA = TB % NW

BLK_E = 2000
GRID_E = E // BLK_E
BLK_N = 2000
GRID_N = N // BLK_N


def _p1_body(rel_ref, w1t_ref, u1c_ref, rel2_ref, rdot_ref, rmax_ref):
    rel = rel_ref[...]
    rel2_ref[...] = jnp.dot(rel, w1t_ref[...])
    rd = jnp.dot(rel, u1c_ref[...])
    rdot_ref[...] = rd
    rmax_ref[...] = jnp.max(rd).reshape(1, 1, 1)


def _p1(rel, w1t, u1c):
    return pl.pallas_call(
        _p1_body,
        grid=(GRID_E,),
        in_specs=[
            pl.BlockSpec((BLK_E, H), lambda i: (i, 0)),
            pl.BlockSpec((H, H), lambda i: (0, 0)),
            pl.BlockSpec((H, 1), lambda i: (0, 0)),
        ],
        out_specs=[
            pl.BlockSpec((BLK_E, H), lambda i: (i, 0)),
            pl.BlockSpec((BLK_E, 1), lambda i: (i, 0)),
            pl.BlockSpec((1, 1, 1), lambda i: (i, 0, 0)),
        ],
        out_shape=[
            jax.ShapeDtypeStruct((E, H), jnp.float32),
            jax.ShapeDtypeStruct((E, 1), jnp.float32),
            jax.ShapeDtypeStruct((GRID_E, 1, 1), jnp.float32),
        ],
    )(rel, w1t, u1c)


def _p1b_body(node_ref, w2t_ref, u2c_ref, nt2_ref, ndot_ref, nmax_ref):
    node = node_ref[...]
    nt2_ref[...] = jnp.dot(node, w2t_ref[...])
    nd = jnp.dot(node, u2c_ref[...])
    ndot_ref[...] = nd
    nmax_ref[...] = jnp.max(nd).reshape(1, 1, 1)


def _p1b(node, w2t, u2c):
    return pl.pallas_call(
        _p1b_body,
        grid=(GRID_N,),
        in_specs=[
            pl.BlockSpec((BLK_N, H), lambda i: (i, 0)),
            pl.BlockSpec((H, H), lambda i: (0, 0)),
            pl.BlockSpec((H, 1), lambda i: (0, 0)),
        ],
        out_specs=[
            pl.BlockSpec((BLK_N, H), lambda i: (i, 0)),
            pl.BlockSpec((BLK_N, 1), lambda i: (i, 0)),
            pl.BlockSpec((1, 1, 1), lambda i: (i, 0, 0)),
        ],
        out_shape=[
            jax.ShapeDtypeStruct((N, H), jnp.float32),
            jax.ShapeDtypeStruct((N, 1), jnp.float32),
            jax.ShapeDtypeStruct((GRID_N, 1, 1), jnp.float32),
        ],
    )(node, w2t, u2c)


B3 = 80
TB3 = E // B3
NB3_BASE = TB3 // NW
NB3_EXTRA = TB3 % NW
NBV3 = 126


def _p23_body(tails_hbm, heads_hbm, rdot_hbm, ndot_hbm, cvt_hbm, shift_hbm,
              rel2_hbm, nt2_hbm,
              sp_hbm, segp_hbm,
              tb, hb, rd, nd, cv, ev, rr, nr, shv, s_sh, g_sh,
              sl0, sl1, sr0, sr1, sn0, sn1, sg0, sg1):
    c = lax.axis_index("c")
    s = lax.axis_index("s")
    w = s * NC + c
    nbr = NB3_BASE + jnp.where(w < NB3_EXTRA, 1, 0)
    base = (NB3_BASE * w + jnp.minimum(w, NB3_EXTRA)) * B3
    sl = [sl0, sl1]
    sr = [sr0, sr1]
    sn = [sn0, sn1]
    sg = [sg0, sg1]

    def off_of(j):
        return base + jnp.minimum(j, nbr - 1) * B3

    zeros16 = jnp.zeros((16,), jnp.float32)

    def zrow_init(r, carry):
        for k in range(H // 16):
            rr[0, r, pl.ds(k * 16, 16)] = zeros16
        return carry

    lax.fori_loop(0, B3, zrow_init, None)
    for g in range(B3 // 16):
        ev[0, pl.ds(g * 16, 16)] = zeros16
    for i in range(640 // B3):
        pltpu.sync_copy(rr.at[0, pl.ds(0, B3)],
                        s_sh.at[pl.ds(s * 640 + i * B3, B3)])
        pltpu.sync_copy(ev.at[0],
                        g_sh.at[pl.ds(s * 640 + i * B3, B3)])
    pltpu.sync_copy(shift_hbm, shv)
    shift = shv[pl.ds(0, 16)][0]
    plsc.subcore_barrier()

    def issue_small(j, q):
        off = off_of(j)
        pltpu.async_copy(tails_hbm.at[pl.ds(off, B3)], tb.at[q], sl[q])
        pltpu.async_copy(heads_hbm.at[pl.ds(off, B3)], hb.at[q], sl[q])
        pltpu.async_copy(rdot_hbm.at[pl.ds(off, B3)], rd.at[q], sl[q])

    def wait_small(j, q):
        off = off_of(j)
        pltpu.make_async_copy(tails_hbm.at[pl.ds(off, B3)], tb.at[q], sl[q]).wait()
        pltpu.make_async_copy(heads_hbm.at[pl.ds(off, B3)], hb.at[q], sl[q]).wait()
        pltpu.make_async_copy(rdot_hbm.at[pl.ds(off, B3)], rd.at[q], sl[q]).wait()

    def issue_gath(q):
        pltpu.async_copy(ndot_hbm.at[hb.at[q]], nd.at[q], sg[q])
        pltpu.async_copy(cvt_hbm.at[tb.at[q]], cv.at[q], sg[q])

    def wait_gath(q):
        pltpu.make_async_copy(ndot_hbm.at[hb.at[q]], nd.at[q], sg[q]).wait()
        pltpu.make_async_copy(cvt_hbm.at[tb.at[q]], cv.at[q], sg[q]).wait()

    def issue_rr(j, q):
        pltpu.async_copy(rel2_hbm.at[pl.ds(off_of(j), B3)], rr.at[q], sr[q])

    def wait_rr(j, q):
        pltpu.make_async_copy(rel2_hbm.at[pl.ds(off_of(j), B3)], rr.at[q],
                              sr[q]).wait()

    def issue_nr(q):
        pltpu.async_copy(nt2_hbm.at[hb.at[q]], nr.at[q], sn[q])

    def wait_nr(q):
        pltpu.make_async_copy(nt2_hbm.at[hb.at[q]], nr.at[q], sn[q]).wait()

    def compute(j, q):
        for g in range(B3 // 16):
            d = pl.ds(g * 16, 16)
            ev[q, d] = jnp.exp(rd[q, d] + nd[q, d] - shift) * cv[q, d]

        @pl.when(j >= nbr)
        def _():
            for g in range(B3 // 16):
                ev[q, pl.ds(g * 16, 16)] = zeros16

        def scale(ri, c2):
            d16 = pl.ds(ri * 16, 16)
            ev16 = ev[q, d16]
            tb[q, d16] = jnp.where(ev16 > 0.0, tb[q, d16], N)
            for li in range(16):
                row = ri * 16 + li
                sc = ev16[li]

                @pl.when(sc > 0.0)
                def _():
                    for k in range(H // 16):
                        d = pl.ds(k * 16, 16)
                        rr[q, row, d] = (rr[q, row, d] + nr[q, row, d]) * sc
            return c2

        lax.fori_loop(0, B3 // 16, scale, None)

    issue_small(0, 0)
    issue_rr(0, 0)
    issue_small(1, 1)
    issue_rr(1, 1)
    wait_small(0, 0)
    issue_gath(0)
    issue_nr(0)

    def pair(i, carry):
        for t in range(2):
            j = i * 2 + t
            q, q1 = t, 1 - t
            wait_small(j + 1, q1)
            issue_gath(q1)
            issue_nr(q1)
            wait_rr(j, q)
            wait_nr(q)
            wait_gath(q)
            compute(j, q)
            pltpu.sync_copy(ev.at[q], g_sh.at[tb.at[q]], add=True)
            pltpu.sync_copy(rr.at[q], s_sh.at[tb.at[q]], add=True)
            issue_small(j + 2, q)
            issue_rr(j + 2, q)
        return carry

    lax.fori_loop(0, NBV3 // 2, pair, None)

    wait_small(NBV3 + 1, 1)
    wait_rr(NBV3, 0)
    wait_rr(NBV3 + 1, 1)
    wait_nr(0)
    wait_gath(0)

    plsc.subcore_barrier()
    pltpu.sync_copy(s_sh.at[pl.ds(s * 640, 640)],
                    sp_hbm.at[c, pl.ds(s * 640, 640)])
    pltpu.sync_copy(g_sh.at[pl.ds(s * 640, 640)],
                    segp_hbm.at[c, pl.ds(s * 640, 640)])


def _p23(tails, heads, rdot, ndot, cvtf, shift128, rel2, nt2):
    mesh = plsc.VectorSubcoreMesh(core_axis_name="c", subcore_axis_name="s",
                                  num_cores=NC, num_subcores=NS)
    f = pl.kernel(
        _p23_body,
        out_type=[
            jax.ShapeDtypeStruct((NC, NPAD, H), jnp.float32),
            jax.ShapeDtypeStruct((NC, NPAD), jnp.float32),
        ],
        mesh=mesh,
        compiler_params=pltpu.CompilerParams(needs_layout_passes=False),
        scratch_types=[
            pltpu.VMEM((2, B3), jnp.int32),
            pltpu.VMEM((2, B3), jnp.int32),
            pltpu.VMEM((2, B3), jnp.float32),
            pltpu.VMEM((2, B3), jnp.float32),
            pltpu.VMEM((2, B3), jnp.float32),
            pltpu.VMEM((2, B3), jnp.float32),
            pltpu.VMEM((2, B3, H), jnp.float32),
            pltpu.VMEM((2, B3, H), jnp.float32),
            pltpu.VMEM((128,), jnp.float32),
            pltpu.VMEM_SHARED((NPAD, H), jnp.float32),
            pltpu.VMEM_SHARED((NPAD,), jnp.float32),
        ] + [pltpu.SemaphoreType.DMA] * 8,
    )
    return f(tails, heads, rdot, ndot, cvtf, shift128, rel2, nt2)


def _p4_body(node_ref, sp_ref, seg_ref, cvt_ref, shc_ref, out_ref):
    seg = seg_ref[0] + seg_ref[1]
    dinv = 1.0 / jnp.where(seg > 0.0, seg, 1.0)
    agg = (sp_ref[0] + sp_ref[1]) * dinv[:, None] + shc_ref[...]
    cv = cvt_ref[0]
    out_ref[...] = jnp.where(cv[:, None] > 0.0, agg, node_ref[...])


BLK_P4 = 2048
GRID_P4 = NPAD // BLK_P4


def _p4(node, sp, segp, cvt2d, shc2d):
    return pl.pallas_call(
        _p4_body,
        grid=(GRID_P4,),
        in_specs=[
            pl.BlockSpec((BLK_P4, H), lambda i: (i, 0)),
            pl.BlockSpec((NC, BLK_P4, H), lambda i: (0, i, 0)),
            pl.BlockSpec((NC, BLK_P4), lambda i: (0, i)),
            pl.BlockSpec((1, BLK_P4), lambda i: (0, i)),
            pl.BlockSpec((1, H), lambda i: (0, 0)),
        ],
        out_specs=pl.BlockSpec((BLK_P4, H), lambda i: (i, 0)),
        out_shape=jax.ShapeDtypeStruct((N, H), jnp.float32),
    )(node, sp, segp, cvt2d, shc2d)


@jax.jit
def kernel(node_tokens, relation_tokens, edge_index, node_is_cvt, W,
           attn_vector, shared_cvt):
    heads = edge_index[0]
    tails = edge_index[1]
    cvtf = node_is_cvt.astype(jnp.float32)
    w1t = W[:, :H].T
    w2t = W[:, H:].T
    u1c = jnp.dot(w1t, attn_vector).reshape(H, 1)
    u2c = jnp.dot(w2t, attn_vector).reshape(H, 1)

    rel2, rdot2, rmax3 = _p1(relation_tokens, w1t, u1c)
    nt2, ndot2, nmax3 = _p1b(node_tokens, w2t, u2c)

    shift = jnp.max(rmax3) + jnp.max(nmax3)
    shift128 = jnp.broadcast_to(shift, (128,))
    rdot = rdot2.reshape(E)
    ndot = ndot2.reshape(N)

    sp, segp = _p23(tails, heads, rdot, ndot, cvtf, shift128, rel2, nt2)

    return _p4(node_tokens, sp, segp, cvtf.reshape(1, N),
               shared_cvt.reshape(1, H))
